# baseline (speedup 1.0000x reference)
import jax
import jax.numpy as jnp
from jax import lax
from jax.experimental import pallas as pl
from jax.experimental.pallas import tpu as pltpu
from jax.experimental.pallas import tpu_sc as plsc

BATCH = 16384
FACTORS = 16
_NC = 2
_NS = 16
_NW = _NC * _NS
_BPW = BATCH // _NW
_L = 16
_NCHUNK = _BPW // _L
_HW = 8
_NH = _BPW // _HW
_RING = 3


def _body(user_hbm, item_hbm, ufT_hbm, ifT_hbm, out_hbm, *s):
    uflat, iflat, out_v = s[0], s[1], s[2]
    uidx_v, iidx_v = s[3], s[4]
    nwin = _RING * _HW
    uwin = tuple(s[5 + r * _HW:5 + (r + 1) * _HW] for r in range(_RING))
    iwin = tuple(s[5 + nwin + r * _HW:5 + nwin + (r + 1) * _HW]
                 for r in range(_RING))
    sem_u = s[5 + 2 * nwin:5 + 2 * nwin + _RING]
    sem_i = s[5 + 2 * nwin + _RING:5 + 2 * nwin + 2 * _RING]

    wid = lax.axis_index("s") * _NC + lax.axis_index("c")
    base = wid * _BPW
    pltpu.sync_copy(user_hbm.at[pl.ds(base, _BPW)],
                    uidx_v.at[pl.ds(0, _BPW)])
    pltpu.sync_copy(item_hbm.at[pl.ds(base, _BPW)],
                    iidx_v.at[pl.ds(0, _BPW)])

    lane = lax.iota(jnp.int32, _L)

    def idx_vecs(h):
        return uidx_v[pl.ds(h * _HW, _L)], iidx_v[pl.ds(h * _HW, _L)]

    def fire(h, r):
        uvec, ivec = idx_vecs(h)
        for j in range(_HW):
            wu = pl.multiple_of((uvec[j] >> 7) * 128, 128)
            wi = pl.multiple_of((ivec[j] >> 7) * 128, 128)
            for t in range(2):
                sub = pl.ds(t * 8, 8)
                pltpu.async_copy(ufT_hbm.at[sub, pl.ds(wu, 128)],
                                 uwin[r][j].at[sub, :], sem_u[r])
                pltpu.async_copy(ifT_hbm.at[sub, pl.ds(wi, 128)],
                                 iwin[r][j].at[sub, :], sem_i[r])

    def drain_extract(h, r):
        for j in range(_HW):
            for t in range(2):
                sub = pl.ds(t * 8, 8)
                pltpu.make_async_copy(ufT_hbm.at[sub, pl.ds(0, 128)],
                                      uwin[r][j].at[sub, :],
                                      sem_u[r]).wait()
                pltpu.make_async_copy(ifT_hbm.at[sub, pl.ds(0, 128)],
                                      iwin[r][j].at[sub, :],
                                      sem_i[r]).wait()
        uvec, ivec = idx_vecs(h)
        for j in range(_HW):
            cu = jnp.full((_L,), uvec[j] & 127, jnp.int32)
            ci = jnp.full((_L,), ivec[j] & 127, jnp.int32)
            u = plsc.load_gather(uwin[r][j], [lane, cu])
            v = plsc.load_gather(iwin[r][j], [lane, ci])
            e = h * _HW + j
            uflat[pl.ds(e * _L, _L)] = u
            iflat[pl.ds(e * _L, _L)] = v

    for r in range(_RING):
        fire(r, r)

    def step(g, carry):
        for k in range(_RING):
            h = _RING * g + k
            drain_extract(h, k)
            fire(h + _RING, k)
        return carry

    ngroups = (_NH - _RING - (_RING - 1)) // _RING
    lax.fori_loop(0, ngroups, step, 0)
    for h in range(_RING * ngroups, _NH - _RING):
        drain_extract(h, h % _RING)
        fire(h + _RING, (h + _RING) % _RING)
    for h in range(_NH - _RING, _NH):
        drain_extract(h, h % _RING)

    def chunk(c, carry):
        idx0 = (c * _L + lane) * FACTORS
        acc = jnp.zeros((_L,), jnp.float32)
        for f in range(FACTORS):
            u = plsc.load_gather(uflat, [idx0 + f])
            v = plsc.load_gather(iflat, [idx0 + f])
            acc = acc + u * v
        out_v[pl.ds(c * _L, _L)] = acc
        return carry

    lax.fori_loop(0, _NCHUNK, chunk, 0)
    pltpu.sync_copy(out_v, out_hbm.at[pl.ds(base, _BPW)])


@jax.jit
def kernel(user, item, user_factor, item_factor):
    mesh = plsc.VectorSubcoreMesh(core_axis_name="c", subcore_axis_name="s")
    scratch = (
        [pltpu.VMEM((_BPW * FACTORS,), jnp.float32)] * 2
        + [pltpu.VMEM((_BPW,), jnp.float32)]
        + [pltpu.VMEM((_BPW + _L,), jnp.int32)] * 2
        + [pltpu.VMEM((FACTORS, 128), jnp.float32)
           for _ in range(2 * _RING * _HW)]
        + [pltpu.SemaphoreType.DMA for _ in range(2 * _RING)]
    )
    k = pl.kernel(
        _body,
        out_type=jax.ShapeDtypeStruct((BATCH,), jnp.float32),
        mesh=mesh,
        compiler_params=pltpu.CompilerParams(
            needs_layout_passes=False, use_tc_tiling_on_sc=True),
        scratch_types=scratch,
    )
    return k(user.astype(jnp.int32), item.astype(jnp.int32),
             user_factor.T, item_factor.T)


# speedup vs baseline: 1.0098x; 1.0098x over previous
import jax
import jax.numpy as jnp
from jax import lax
from jax.experimental import pallas as pl
from jax.experimental.pallas import tpu as pltpu
from jax.experimental.pallas import tpu_sc as plsc

BATCH = 16384
FACTORS = 16
_NC = 2
_NS = 16
_NW = _NC * _NS
_BPW = BATCH // _NW
_L = 16
_NCHUNK = _BPW // _L
_HW = 8
_NH = _BPW // _HW
_RING = 3


def _body(user_hbm, item_hbm, ufT_hbm, ifT_hbm, out_hbm, *s):
    uflat, iflat, out_v = s[0], s[1], s[2]
    uidx_v, iidx_v = s[3], s[4]
    nwin = _RING * _HW
    uwin = tuple(s[5 + r * _HW:5 + (r + 1) * _HW] for r in range(_RING))
    iwin = tuple(s[5 + nwin + r * _HW:5 + nwin + (r + 1) * _HW]
                 for r in range(_RING))
    sem_u = s[5 + 2 * nwin:5 + 2 * nwin + _RING]
    sem_i = s[5 + 2 * nwin + _RING:5 + 2 * nwin + 2 * _RING]

    wid = lax.axis_index("s") * _NC + lax.axis_index("c")
    base = wid * _BPW
    pltpu.sync_copy(user_hbm.at[pl.ds(base, _BPW)],
                    uidx_v.at[pl.ds(0, _BPW)])
    pltpu.sync_copy(item_hbm.at[pl.ds(base, _BPW)],
                    iidx_v.at[pl.ds(0, _BPW)])

    lane = lax.iota(jnp.int32, _L)

    def idx_vecs(h):
        return uidx_v[pl.ds(h * _HW, _L)], iidx_v[pl.ds(h * _HW, _L)]

    def fire(h, r):
        uvec, ivec = idx_vecs(h)
        for j in range(_HW):
            wu = pl.multiple_of((uvec[j] >> 7) * 128, 128)
            wi = pl.multiple_of((ivec[j] >> 7) * 128, 128)
            pltpu.async_copy(ufT_hbm.at[:, pl.ds(wu, 128)], uwin[r][j],
                             sem_u[r])
            pltpu.async_copy(ifT_hbm.at[:, pl.ds(wi, 128)], iwin[r][j],
                             sem_i[r])

    def drain_extract(h, r):
        for j in range(_HW):
            pltpu.make_async_copy(ufT_hbm.at[:, pl.ds(0, 128)], uwin[r][j],
                                  sem_u[r]).wait()
            pltpu.make_async_copy(ifT_hbm.at[:, pl.ds(0, 128)], iwin[r][j],
                                  sem_i[r]).wait()
        uvec, ivec = idx_vecs(h)
        for j in range(_HW):
            cu = jnp.full((_L,), uvec[j] & 127, jnp.int32)
            ci = jnp.full((_L,), ivec[j] & 127, jnp.int32)
            u = plsc.load_gather(uwin[r][j], [lane, cu])
            v = plsc.load_gather(iwin[r][j], [lane, ci])
            e = h * _HW + j
            uflat[pl.ds(e * _L, _L)] = u
            iflat[pl.ds(e * _L, _L)] = v

    for r in range(_RING):
        fire(r, r)

    def step(g, carry):
        for k in range(_RING):
            h = _RING * g + k
            drain_extract(h, k)
            fire(h + _RING, k)
        return carry

    ngroups = (_NH - _RING - (_RING - 1)) // _RING
    lax.fori_loop(0, ngroups, step, 0)
    for h in range(_RING * ngroups, _NH - _RING):
        drain_extract(h, h % _RING)
        fire(h + _RING, (h + _RING) % _RING)
    for h in range(_NH - _RING, _NH):
        drain_extract(h, h % _RING)

    def chunk(c, carry):
        idx0 = (c * _L + lane) * FACTORS
        acc = jnp.zeros((_L,), jnp.float32)
        for f in range(FACTORS):
            u = plsc.load_gather(uflat, [idx0 + f])
            v = plsc.load_gather(iflat, [idx0 + f])
            acc = acc + u * v
        out_v[pl.ds(c * _L, _L)] = acc
        return carry

    lax.fori_loop(0, _NCHUNK, chunk, 0)
    pltpu.sync_copy(out_v, out_hbm.at[pl.ds(base, _BPW)])


@jax.jit
def kernel(user, item, user_factor, item_factor):
    mesh = plsc.VectorSubcoreMesh(core_axis_name="c", subcore_axis_name="s")
    scratch = (
        [pltpu.VMEM((_BPW * FACTORS,), jnp.float32)] * 2
        + [pltpu.VMEM((_BPW,), jnp.float32)]
        + [pltpu.VMEM((_BPW + _L,), jnp.int32)] * 2
        + [pltpu.VMEM((FACTORS, 128), jnp.float32)
           for _ in range(2 * _RING * _HW)]
        + [pltpu.SemaphoreType.DMA for _ in range(2 * _RING)]
    )
    k = pl.kernel(
        _body,
        out_type=jax.ShapeDtypeStruct((BATCH,), jnp.float32),
        mesh=mesh,
        compiler_params=pltpu.CompilerParams(
            needs_layout_passes=False, use_tc_tiling_on_sc=True),
        scratch_types=scratch,
    )
    return k(user.astype(jnp.int32), item.astype(jnp.int32),
             user_factor.T, item_factor.T)
